# baseline (speedup 1.0000x reference)
import functools

import jax
import jax.numpy as jnp
from jax import lax
from jax.experimental import pallas as pl
from jax.experimental.pallas import tpu as pltpu

C = 128


def _body(map_ref, act_ref, x_ref, w1_ref, b1_ref, w2_ref, b2_ref,
          tok_out_ref, act_out_ref, M, D, A, F, *, tn, nt, h):
    t = pl.program_id(1)

    @pl.when(t == 0)
    def _init():
        M[...] = jnp.full((C, 1), -1e30, jnp.float32)
        D[...] = jnp.zeros((C, 1), jnp.float32)
        A[...] = jnp.zeros((C, h), jnp.float32)
        F[...] = jnp.zeros((C, 1), jnp.float32)

    x = x_ref[0]
    chan_row = map_ref[0]
    act_row = act_ref[0]

    h_t = jnp.tanh(
        lax.dot_general(w1_ref[...], x, (((1,), (1,)), ((), ())),
                        preferred_element_type=jnp.float32)
        + b1_ref[...])
    g_row = (jnp.dot(w2_ref[...], h_t, preferred_element_type=jnp.float32)
             + b2_ref[...])

    ci = lax.broadcasted_iota(jnp.int32, (C, tn), 0)
    onehot = (ci == chan_row) & (act_row > 0.0)

    g_c = jnp.where(onehot, g_row, -1e30)
    m_t = jnp.max(g_c, axis=1, keepdims=True)
    m_new = jnp.maximum(M[...], m_t)
    alpha = jnp.exp(M[...] - m_new)
    p = jnp.where(onehot, jnp.exp(g_row - m_new), 0.0)
    d_t = jnp.sum(p, axis=1, keepdims=True)

    M[...] = m_new
    D[...] = D[...] * alpha + d_t
    A[...] = A[...] * alpha + jnp.dot(p, x, preferred_element_type=jnp.float32)
    F[...] = jnp.maximum(F[...], jnp.max(onehot.astype(jnp.float32), axis=1,
                                         keepdims=True))

    @pl.when(t == nt - 1)
    def _finish():
        tok_out_ref[0] = A[...] / jnp.maximum(D[...], 1e-30)
        act_out_ref[0] = F[...]


def kernel(block_tokens, block_active, block_to_channel_map, W1, b1, W2, b2):
    B, NB, H = block_tokens.shape
    K = W1.shape[0]
    TN = 512
    NT = NB // TN

    map3 = block_to_channel_map.astype(jnp.int32).reshape(1, 1, NB)
    act3 = block_active.astype(jnp.float32).reshape(B, 1, NB)
    b1c = b1.reshape(K, 1)
    b2c = jnp.asarray(b2).reshape(1, 1)

    grid = (B, NT)
    out_tok, out_act = pl.pallas_call(
        functools.partial(_body, tn=TN, nt=NT, h=H),
        grid=grid,
        in_specs=[
            pl.BlockSpec((1, 1, TN), lambda b, t: (0, 0, t)),
            pl.BlockSpec((1, 1, TN), lambda b, t: (b, 0, t)),
            pl.BlockSpec((1, TN, H), lambda b, t: (b, t, 0)),
            pl.BlockSpec((K, H), lambda b, t: (0, 0)),
            pl.BlockSpec((K, 1), lambda b, t: (0, 0)),
            pl.BlockSpec((1, K), lambda b, t: (0, 0)),
            pl.BlockSpec((1, 1), lambda b, t: (0, 0)),
        ],
        out_specs=[
            pl.BlockSpec((1, C, H), lambda b, t: (b, 0, 0)),
            pl.BlockSpec((1, C, 1), lambda b, t: (b, 0, 0)),
        ],
        out_shape=[
            jax.ShapeDtypeStruct((B, C, H), jnp.float32),
            jax.ShapeDtypeStruct((B, C, 1), jnp.float32),
        ],
        scratch_shapes=[
            pltpu.VMEM((C, 1), jnp.float32),
            pltpu.VMEM((C, 1), jnp.float32),
            pltpu.VMEM((C, H), jnp.float32),
            pltpu.VMEM((C, 1), jnp.float32),
        ],
        compiler_params=pltpu.CompilerParams(
            dimension_semantics=("parallel", "arbitrary")),
    )(map3, act3, block_tokens, W1, b1c, W2, b2c)

    return out_tok, out_act.reshape(B, C) > 0.0


# speedup vs baseline: 1.1093x; 1.1093x over previous
import functools

import jax
import jax.numpy as jnp
from jax import lax
from jax.experimental import pallas as pl
from jax.experimental.pallas import tpu as pltpu

C = 128


def _body(map_ref, act_ref, x_ref, w1_ref, b1_ref, w2_ref, b2_ref,
          tok_out_ref, act_out_ref, D, A, *, tn, nt, h):
    t = pl.program_id(1)

    @pl.when(t == 0)
    def _init():
        D[...] = jnp.zeros((C, 1), jnp.float32)
        A[...] = jnp.zeros((C, h), jnp.float32)

    x = x_ref[0]
    chan_row = map_ref[0]
    act_row = act_ref[0]

    h_t = jnp.tanh(
        lax.dot_general(w1_ref[...], x, (((1,), (1,)), ((), ())),
                        preferred_element_type=jnp.float32)
        + b1_ref[...])
    g_row = (jnp.dot(w2_ref[...], h_t, preferred_element_type=jnp.float32)
             + b2_ref[...])
    p_row = jnp.exp(jnp.clip(g_row, -40.0, 40.0)) * act_row

    ci = lax.broadcasted_iota(jnp.int32, (C, tn), 0)
    p = jnp.where(chan_row == ci, p_row, 0.0)

    D[...] += jnp.sum(p, axis=1, keepdims=True)
    A[...] += jnp.dot(p, x, preferred_element_type=jnp.float32)

    @pl.when(t == nt - 1)
    def _finish():
        d = D[...]
        tok_out_ref[0] = A[...] / jnp.maximum(d, 1e-30)
        act_out_ref[0] = (d > 0.0).astype(jnp.float32)


def kernel(block_tokens, block_active, block_to_channel_map, W1, b1, W2, b2):
    B, NB, H = block_tokens.shape
    K = W1.shape[0]
    TN = 512
    NT = NB // TN

    map3 = block_to_channel_map.astype(jnp.int32).reshape(1, 1, NB)
    act3 = block_active.astype(jnp.float32).reshape(B, 1, NB)
    b1c = b1.reshape(K, 1)
    b2c = jnp.asarray(b2).reshape(1, 1)

    grid = (B, NT)
    out_tok, out_act = pl.pallas_call(
        functools.partial(_body, tn=TN, nt=NT, h=H),
        grid=grid,
        in_specs=[
            pl.BlockSpec((1, 1, TN), lambda b, t: (0, 0, t)),
            pl.BlockSpec((1, 1, TN), lambda b, t: (b, 0, t)),
            pl.BlockSpec((1, TN, H), lambda b, t: (b, t, 0)),
            pl.BlockSpec((K, H), lambda b, t: (0, 0)),
            pl.BlockSpec((K, 1), lambda b, t: (0, 0)),
            pl.BlockSpec((1, K), lambda b, t: (0, 0)),
            pl.BlockSpec((1, 1), lambda b, t: (0, 0)),
        ],
        out_specs=[
            pl.BlockSpec((1, C, H), lambda b, t: (b, 0, 0)),
            pl.BlockSpec((1, C, 1), lambda b, t: (b, 0, 0)),
        ],
        out_shape=[
            jax.ShapeDtypeStruct((B, C, H), jnp.float32),
            jax.ShapeDtypeStruct((B, C, 1), jnp.float32),
        ],
        scratch_shapes=[
            pltpu.VMEM((C, 1), jnp.float32),
            pltpu.VMEM((C, H), jnp.float32),
        ],
        compiler_params=pltpu.CompilerParams(
            dimension_semantics=("parallel", "arbitrary")),
    )(map3, act3, block_tokens, W1, b1c, W2, b2c)

    return out_tok, out_act.reshape(B, C) > 0.0


# speedup vs baseline: 1.7776x; 1.6025x over previous
import functools

import jax
import jax.numpy as jnp
from jax import lax
from jax.experimental import pallas as pl
from jax.experimental.pallas import tpu as pltpu

C = 128


def _body(map_ref, act_ref, x_ref, w1_ref, b1_ref, w2_ref, b2_ref,
          tok_out_ref, act_out_ref, D, A, *, tn, nt, h, ns):
    t = pl.program_id(1)

    @pl.when(t == 0)
    def _init():
        D[...] = jnp.zeros((C, 1), jnp.float32)
        A[...] = jnp.zeros((C, h), jnp.float32)

    sn = tn // ns
    ci = lax.broadcasted_iota(jnp.int32, (C, sn), 0)
    d_parts = []
    a_parts = []
    for s in range(ns):
        x = x_ref[0, pl.ds(s * sn, sn), :]
        chan_row = map_ref[0, :, pl.ds(s * sn, sn)]
        act_row = act_ref[0, :, pl.ds(s * sn, sn)]

        h_t = jnp.tanh(
            lax.dot_general(w1_ref[...], x, (((1,), (1,)), ((), ())),
                            preferred_element_type=jnp.float32)
            + b1_ref[...])
        g_row = (jnp.dot(w2_ref[...], h_t, preferred_element_type=jnp.float32)
                 + b2_ref[...])
        p_row = jnp.exp(jnp.clip(g_row, -40.0, 40.0)) * act_row

        p = jnp.where(chan_row == ci, p_row, 0.0)

        d_parts.append(jnp.sum(p, axis=1, keepdims=True))
        a_parts.append(jnp.dot(p, x, preferred_element_type=jnp.float32))

    D[...] += sum(d_parts)
    A[...] += sum(a_parts)

    @pl.when(t == nt - 1)
    def _finish():
        d = D[...]
        tok_out_ref[0] = A[...] / jnp.maximum(d, 1e-30)
        act_out_ref[0] = (d > 0.0).astype(jnp.float32)


def kernel(block_tokens, block_active, block_to_channel_map, W1, b1, W2, b2):
    B, NB, H = block_tokens.shape
    K = W1.shape[0]
    TN = 1024
    NS = 2
    NT = NB // TN

    map3 = block_to_channel_map.astype(jnp.int32).reshape(1, 1, NB)
    act3 = block_active.astype(jnp.float32).reshape(B, 1, NB)
    b1c = b1.reshape(K, 1)
    b2c = jnp.asarray(b2).reshape(1, 1)

    grid = (B, NT)
    out_tok, out_act = pl.pallas_call(
        functools.partial(_body, tn=TN, nt=NT, h=H, ns=NS),
        grid=grid,
        in_specs=[
            pl.BlockSpec((1, 1, TN), lambda b, t: (0, 0, t)),
            pl.BlockSpec((1, 1, TN), lambda b, t: (b, 0, t)),
            pl.BlockSpec((1, TN, H), lambda b, t: (b, t, 0)),
            pl.BlockSpec((K, H), lambda b, t: (0, 0)),
            pl.BlockSpec((K, 1), lambda b, t: (0, 0)),
            pl.BlockSpec((1, K), lambda b, t: (0, 0)),
            pl.BlockSpec((1, 1), lambda b, t: (0, 0)),
        ],
        out_specs=[
            pl.BlockSpec((1, C, H), lambda b, t: (b, 0, 0)),
            pl.BlockSpec((1, C, 1), lambda b, t: (b, 0, 0)),
        ],
        out_shape=[
            jax.ShapeDtypeStruct((B, C, H), jnp.float32),
            jax.ShapeDtypeStruct((B, C, 1), jnp.float32),
        ],
        scratch_shapes=[
            pltpu.VMEM((C, 1), jnp.float32),
            pltpu.VMEM((C, H), jnp.float32),
        ],
        compiler_params=pltpu.CompilerParams(
            dimension_semantics=("parallel", "arbitrary")),
    )(map3, act3, block_tokens, W1, b1c, W2, b2c)

    return out_tok, out_act.reshape(B, C) > 0.0


# speedup vs baseline: 2.2006x; 1.2379x over previous
import functools

import jax
import jax.numpy as jnp
from jax import lax
from jax.experimental import pallas as pl
from jax.experimental.pallas import tpu as pltpu

C = 128


def _body(map_ref, act_ref, x_ref, w1_ref, b1_ref, w2_ref, b2_ref,
          tok_out_ref, act_out_ref, D, A, *, tn, nt, h, ns):
    t = pl.program_id(1)

    @pl.when(t == 0)
    def _init():
        D[...] = jnp.zeros((C, 1), jnp.float32)
        A[...] = jnp.zeros((C, h), jnp.float32)

    sn = tn // ns
    ci = lax.broadcasted_iota(jnp.int32, (C, sn), 0)
    d_parts = []
    a_parts = []
    for s in range(ns):
        x = x_ref[0, pl.ds(s * sn, sn), :]
        chan_row = map_ref[0, :, pl.ds(s * sn, sn)]
        act_row = act_ref[0, :, pl.ds(s * sn, sn)]

        h_t = jnp.tanh(
            lax.dot_general(w1_ref[...], x, (((1,), (1,)), ((), ())),
                            preferred_element_type=jnp.float32)
            + b1_ref[...])
        g_row = (jnp.dot(w2_ref[...], h_t, preferred_element_type=jnp.float32)
                 + b2_ref[...])
        p_row = jnp.exp(jnp.clip(g_row, -40.0, 40.0)) * act_row

        p = jnp.where(chan_row == ci, p_row, 0.0)

        d_parts.append(jnp.sum(p, axis=1, keepdims=True))
        a_parts.append(jnp.dot(p, x, preferred_element_type=jnp.float32))

    D[...] += sum(d_parts)
    A[...] += sum(a_parts)

    @pl.when(t == nt - 1)
    def _finish():
        d = D[...]
        tok_out_ref[0] = A[...] / jnp.maximum(d, 1e-30)
        act_out_ref[0] = (d > 0.0).astype(jnp.float32)


def kernel(block_tokens, block_active, block_to_channel_map, W1, b1, W2, b2):
    B, NB, H = block_tokens.shape
    K = W1.shape[0]
    TN = 2048
    NS = 4
    NT = NB // TN

    map3 = block_to_channel_map.astype(jnp.int32).reshape(1, 1, NB)
    act3 = block_active.astype(jnp.float32).reshape(B, 1, NB)
    b1c = b1.reshape(K, 1)
    b2c = jnp.asarray(b2).reshape(1, 1)

    grid = (B, NT)
    out_tok, out_act = pl.pallas_call(
        functools.partial(_body, tn=TN, nt=NT, h=H, ns=NS),
        grid=grid,
        in_specs=[
            pl.BlockSpec((1, 1, TN), lambda b, t: (0, 0, t)),
            pl.BlockSpec((1, 1, TN), lambda b, t: (b, 0, t)),
            pl.BlockSpec((1, TN, H), lambda b, t: (b, t, 0)),
            pl.BlockSpec((K, H), lambda b, t: (0, 0)),
            pl.BlockSpec((K, 1), lambda b, t: (0, 0)),
            pl.BlockSpec((1, K), lambda b, t: (0, 0)),
            pl.BlockSpec((1, 1), lambda b, t: (0, 0)),
        ],
        out_specs=[
            pl.BlockSpec((1, C, H), lambda b, t: (b, 0, 0)),
            pl.BlockSpec((1, C, 1), lambda b, t: (b, 0, 0)),
        ],
        out_shape=[
            jax.ShapeDtypeStruct((B, C, H), jnp.float32),
            jax.ShapeDtypeStruct((B, C, 1), jnp.float32),
        ],
        scratch_shapes=[
            pltpu.VMEM((C, 1), jnp.float32),
            pltpu.VMEM((C, H), jnp.float32),
        ],
        compiler_params=pltpu.CompilerParams(
            dimension_semantics=("parallel", "arbitrary")),
    )(map3, act3, block_tokens, W1, b1c, W2, b2c)

    return out_tok, out_act.reshape(B, C) > 0.0
